# baseline (speedup 1.0000x reference)
import functools

import jax
import jax.numpy as jnp
import numpy as np
from jax import lax
from jax.experimental import pallas as pl
from jax.experimental.pallas import tpu as pltpu
from jax.experimental.pallas import tpu_sc as plsc

F = 128
N_RBF = 32
CUTOFF = 5.0
N_MODULES = 5
LOG2 = 0.6931471805599453

NC = 2
NS = 16
NW = NC * NS
CHUNK = 80


def _ssp(x):
    xx = jnp.abs(x) * -1.4426950408889634
    n = jnp.floor(xx + 0.5)
    fr = xx - n
    p = 1.535336188319500e-4
    p = p * fr + 1.339887440266574e-3
    p = p * fr + 9.618437357674640e-3
    p = p * fr + 5.550332471162809e-2
    p = p * fr + 2.402264791363012e-1
    p = p * fr + 6.931472028550421e-1
    p = p * fr + 1.0
    ei = jnp.clip(n, -127.0, 127.0).astype(jnp.int32)
    t = lax.bitcast_convert_type((ei + 127) << 23, jnp.float32) * p
    y = t / (2.0 + t)
    y2 = y * y
    l = 2.0 * y * (1.0 + y2 * (1.0 / 3.0 + y2 * (0.2 + y2 * (1.0 / 7.0))))
    return jnp.maximum(x, 0.0) + l - LOG2


def _rbf_gate_body(r_ref, wg_ref, g5_ref):
    r = r_ref[...]
    d2 = jnp.sum(r * r, axis=1, keepdims=True)
    d = jnp.sqrt(d2)
    step = CUTOFF / (N_RBF - 1)
    centers = lax.broadcasted_iota(jnp.int32, (1, N_RBF), 1).astype(jnp.float32) * step
    width = CUTOFF / N_RBF
    gamma = 0.5 / (width * width)
    nl2g = -gamma * 1.4426950408889634
    xx = nl2g * (d - centers) ** 2
    n = jnp.floor(xx + 0.5)
    fr = xx - n
    p = 1.535336188319500e-4
    p = p * fr + 1.339887440266574e-3
    p = p * fr + 9.618437357674640e-3
    p = p * fr + 5.550332471162809e-2
    p = p * fr + 2.402264791363012e-1
    p = p * fr + 6.931472028550421e-1
    p = p * fr + 1.0
    ei = jnp.clip(n, -127.0, 127.0).astype(jnp.int32)
    scale = lax.bitcast_convert_type((ei + 127) << 23, jnp.float32)
    phi = scale * p
    dc = jnp.clip(d, 0.0, CUTOFF)
    y = (dc * (jnp.pi / CUTOFF)) - (0.5 * jnp.pi)
    y2 = y * y
    sy = -1.9515295891e-4
    sy = sy * y2 + 8.3321608736e-3
    sy = sy * y2 - 1.6666654611e-1
    sy = sy * y2 + 1.0
    sy = sy * y
    fcut = 0.5 * (1.0 - sy)
    fcut = fcut * (d < CUTOFF).astype(jnp.float32)
    g = phi * fcut
    for m in range(N_MODULES):
        gm = jnp.dot(g, wg_ref[m], preferred_element_type=jnp.float32)
        u = lax.bitcast_convert_type(gm, jnp.uint32)
        half = u.shape[0] // 2
        w = (u[:half] >> 16) | (u[half:] & jnp.uint32(0xFFFF0000))
        g5_ref[m] = lax.bitcast_convert_type(w, jnp.int32)


def _rbf_gate(r_ij, wg5):
    e = r_ij.shape[0]
    blk = 1600
    grid = e // blk
    return pl.pallas_call(
        _rbf_gate_body,
        grid=(grid,),
        in_specs=[
            pl.BlockSpec((blk, 3), lambda i: (i, 0)),
            pl.BlockSpec((N_MODULES, N_RBF, F), lambda i: (0, 0, 0)),
        ],
        out_specs=pl.BlockSpec((N_MODULES, blk // 2, F), lambda i: (0, i, 0)),
        out_shape=jax.ShapeDtypeStruct((N_MODULES, e // 2, F), jnp.int32),
    )(r_ij, wg5)


def _embed_body(tab_hbm, idx_hbm, out_hbm, idx_v, rows_v, sem):
    c = lax.axis_index("c")
    s = lax.axis_index("s")
    wid = c * NS + s
    bpw = out_hbm.shape[0] // NW
    base = wid * bpw
    for k in range(bpw // CHUNK):
        off = base + k * CHUNK
        pltpu.sync_copy(idx_hbm.at[pl.ds(off, CHUNK)], idx_v)
        pltpu.async_copy(tab_hbm.at[idx_v], rows_v, sem).wait()
        pltpu.sync_copy(rows_v, out_hbm.at[pl.ds(off, CHUNK)])


def _embed_gather(table, idx_pad):
    a_pad = idx_pad.shape[0]
    mesh = plsc.VectorSubcoreMesh(core_axis_name="c", subcore_axis_name="s")
    fn = pl.kernel(
        _embed_body,
        out_type=jax.ShapeDtypeStruct((a_pad, F), jnp.float32),
        mesh=mesh,
        scratch_types=[
            pltpu.VMEM((CHUNK,), jnp.int32),
            pltpu.VMEM((CHUNK, F), jnp.float32),
            pltpu.SemaphoreType.DMA,
        ],
    )
    return fn(table, idx_pad)


RING = 2


def _edge_body(g_hbm, mj_hbm, ii_hbm, ij_hbm, z_hbm, out_hbm, *scr):
    ii = scr[0:RING]
    ij = scr[RING:2 * RING]
    gb = scr[2 * RING:3 * RING]
    pb = scr[3 * RING:4 * RING]
    acc = scr[4 * RING]
    sems = scr[4 * RING + 1:]
    sii = sems[0:RING]
    sij = sems[RING:2 * RING]
    sg = sems[2 * RING:3 * RING]
    sm = sems[3 * RING:4 * RING]
    ss = sems[4 * RING:5 * RING]

    c = lax.axis_index("c")
    s = lax.axis_index("s")
    wid = c * NS + s
    a_pad = acc.shape[0]
    rpt = a_pad // NS
    epw = ii_hbm.shape[0] // NW
    nch = epw // CHUNK
    base0 = wid * epw

    pltpu.sync_copy(z_hbm, acc.at[pl.ds(s * rpt, rpt)])
    plsc.subcore_barrier()

    def start(i, b):
        base = base0 + i * CHUNK
        pltpu.async_copy(ii_hbm.at[pl.ds(base, CHUNK)], ii[b], sii[b])
        pltpu.async_copy(ij_hbm.at[pl.ds(base, CHUNK)], ij[b], sij[b])
        gbase = pl.multiple_of(base // 2, 8)
        pltpu.async_copy(g_hbm.at[pl.ds(gbase, CHUNK // 2)], gb[b], sg[b])

    def fire(b):
        pltpu.make_async_copy(ij_hbm.at[pl.ds(0, CHUNK)], ij[b], sij[b]).wait()
        pltpu.async_copy(mj_hbm.at[ij[b]], pb[b], sm[b])

    def wait_scatter(b):
        pltpu.make_async_copy(pb[b], acc.at[ii[b]], ss[b]).wait()

    def proc(b):
        pltpu.make_async_copy(g_hbm.at[pl.ds(0, CHUNK // 2)], gb[b], sg[b]).wait()
        pltpu.make_async_copy(mj_hbm.at[ij[b]], pb[b], sm[b]).wait()
        pltpu.make_async_copy(ii_hbm.at[pl.ds(0, CHUNK)], ii[b], sii[b]).wait()

        @plsc.parallel_loop(0, CHUNK // 2, 1, unroll=8)
        def row_fn(r2):
            e = r2 * 2
            for jj in range(F // 16):
                sl = pl.ds(16 * jj, 16)
                wv = gb[b][r2, sl]
                ga = lax.bitcast_convert_type(
                    lax.shift_left(wv, 16), jnp.float32)
                gc = lax.bitcast_convert_type(
                    lax.bitwise_and(wv, jnp.int32(-65536)), jnp.float32)
                pb[b][e, sl] = ga * pb[b][e, sl]
                pb[b][e + 1, sl] = gc * pb[b][e + 1, sl]
        pltpu.async_copy(pb[b], acc.at[ii[b]], ss[b], add=True)

    start(0, 0)
    fire(0)
    start(1, 1)
    fire(1)
    proc(0)

    def step(i, carry):
        b = lax.rem(i, 2)

        def do(b, o):
            wait_scatter(o)
            start(i + 1, o)
            fire(o)
            proc(b)
        lax.cond(b == 0, lambda: do(0, 1), lambda: do(1, 0))
        return carry
    lax.fori_loop(1, nch - 1, step, 0)

    bl = (nch - 1) % 2
    wait_scatter(1 - bl)
    proc(bl)
    wait_scatter(bl)

    plsc.subcore_barrier()
    pltpu.sync_copy(acc.at[pl.ds(s * rpt, rpt)],
                    out_hbm.at[c, pl.ds(s * rpt, rpt)])


def _edge_stage(g, mj_table, idx_i, idx_j, zrows):
    a_pad = mj_table.shape[0]
    mesh = plsc.VectorSubcoreMesh(core_axis_name="c", subcore_axis_name="s")
    fn = pl.kernel(
        _edge_body,
        out_type=jax.ShapeDtypeStruct((NC, a_pad, F), jnp.float32),
        mesh=mesh,
        scratch_types=(
            [pltpu.VMEM((CHUNK,), jnp.int32) for _ in range(2 * RING)]
            + [pltpu.VMEM((CHUNK // 2, F), jnp.int32) for _ in range(RING)]
            + [pltpu.VMEM((CHUNK, F), jnp.float32) for _ in range(RING)]
            + [pltpu.VMEM_SHARED((a_pad, F), jnp.float32)]
            + [pltpu.SemaphoreType.DMA for _ in range(5 * RING)]
        ),
    )
    return fn(g, mj_table, idx_i, idx_j, zrows)


def _pre_body(x_ref, w_ref, b_ref, mi_ref, mj_ref):
    xt = _ssp(x_ref[...])
    mi_ref[...] = jnp.dot(xt, w_ref[0], preferred_element_type=jnp.float32) + b_ref[0]
    mj_ref[...] = jnp.dot(xt, w_ref[1], preferred_element_type=jnp.float32) + b_ref[1]


def _pre(x, w2, b2):
    a_pad = x.shape[0]
    blk = 512
    grid = a_pad // blk
    sds = jax.ShapeDtypeStruct((a_pad, F), jnp.float32)
    return pl.pallas_call(
        _pre_body,
        grid=(grid,),
        in_specs=[
            pl.BlockSpec((blk, F), lambda i: (i, 0)),
            pl.BlockSpec((2, F, F), lambda i: (0, 0, 0)),
            pl.BlockSpec((2, 1, F), lambda i: (0, 0, 0)),
        ],
        out_specs=[pl.BlockSpec((blk, F), lambda i: (i, 0))] * 2,
        out_shape=[sds, sds],
    )(x, w2, b2)


def _post_body(final, x_ref, mi_ref, agg_ref, w_ref, b_ref, u_ref, *outs):
    def d(h, k):
        return jnp.dot(_ssp(h), w_ref[k], preferred_element_type=jnp.float32) + b_ref[k]

    v = mi_ref[...] + agg_ref[0] + agg_ref[1]
    k = 0
    for _ in range(3):
        v = v + d(d(v, k), k + 1)
        k += 2
    v = d(v, k)
    k += 1
    x = u_ref[...] * x_ref[...] + v
    for _ in range(2):
        x = x + d(d(x, k), k + 1)
        k += 2
    xo = x
    for _ in range(1):
        xo = xo + d(d(xo, k), k + 1)
        k += 2
    outs[0][...] = d(xo, k)
    k += 1
    if not final:
        outs[1][...] = x
        xt = _ssp(x)
        outs[2][...] = jnp.dot(xt, w_ref[k], preferred_element_type=jnp.float32) + b_ref[k]
        outs[3][...] = jnp.dot(xt, w_ref[k + 1], preferred_element_type=jnp.float32) + b_ref[k + 1]


def _post(x, mi, aggp, wstack, bstack, u, final):
    a_pad = x.shape[0]
    blk = 512
    grid = a_pad // blk
    nw = wstack.shape[0]
    sds = jax.ShapeDtypeStruct((a_pad, F), jnp.float32)
    out_shape = [sds] if final else [sds, sds, sds, sds]
    xspec = pl.BlockSpec((blk, F), lambda i: (i, 0))
    return pl.pallas_call(
        functools.partial(_post_body, final),
        grid=(grid,),
        in_specs=[
            xspec, xspec,
            pl.BlockSpec((2, blk, F), lambda i: (0, i, 0)),
            pl.BlockSpec((nw, F, F), lambda i: (0, 0, 0)),
            pl.BlockSpec((nw, 1, F), lambda i: (0, 0, 0)),
            pl.BlockSpec((1, F), lambda i: (0, 0)),
        ],
        out_specs=[xspec] * len(out_shape),
        out_shape=out_shape,
    )(x, mi, aggp, wstack, bstack, u)


def _stack_post_weights(params, m):
    p = params[m]
    ws, bs = [], []
    for rp in p['int_res']:
        ws += [rp['W1'], rp['W2']]
        bs += [rp['b1'], rp['b2']]
    ws.append(p['Wv'])
    bs.append(p['bv'])
    for rp in p['atom_res']:
        ws += [rp['W1'], rp['W2']]
        bs += [rp['b1'], rp['b2']]
    for rp in p['out_res']:
        ws += [rp['W1'], rp['W2']]
        bs += [rp['b1'], rp['b2']]
    ws.append(p['Wout'])
    bs.append(p['bout'])
    if m + 1 < len(params):
        ws += [params[m + 1]['Wi'], params[m + 1]['Wj']]
        bs += [params[m + 1]['bi'], params[m + 1]['bj']]
    wstack = jnp.stack(ws)
    bstack = jnp.stack(bs)[:, None, :]
    return wstack, bstack, p['u'][None, :]


def kernel(atomic_numbers, r_ij, idx_i, idx_j, idx_m, embedding, params):
    n_atoms = atomic_numbers.shape[0]
    a_pad = ((n_atoms + NW * CHUNK - 1) // (NW * CHUNK)) * (NW * CHUNK)

    an_pad = jnp.concatenate([
        atomic_numbers.astype(jnp.int32),
        jnp.zeros((a_pad - n_atoms,), jnp.int32),
    ])
    n_edges = idx_i.shape[0]
    eblk = 1600
    _rr = np.arange(eblk // 2)
    _bases = (np.arange(n_edges // eblk) * eblk)[:, None]
    _inter = np.empty((n_edges // eblk, eblk), np.int32)
    _inter[:, 0::2] = _bases + _rr
    _inter[:, 1::2] = _bases + eblk // 2 + _rr
    eperm = _inter.reshape(-1)
    idx_i = idx_i.astype(jnp.int32)[eperm]
    idx_j = idx_j.astype(jnp.int32)[eperm]
    zrows = jnp.zeros((a_pad // NS, F), jnp.float32)

    wg5 = jnp.stack([p['Wg'] for p in params])
    g5 = _rbf_gate(r_ij, wg5)

    x = _embed_gather(embedding, an_pad)
    w2 = jnp.stack([params[0]['Wi'], params[0]['Wj']])
    b2 = jnp.stack([params[0]['bi'], params[0]['bj']])[:, None, :]
    mi, mj_table = _pre(x, w2, b2)

    outs = []
    for m in range(N_MODULES):
        aggp = _edge_stage(g5[m], mj_table, idx_i, idx_j, zrows)
        final = m == N_MODULES - 1
        wstack, bstack, u = _stack_post_weights(params, m)
        res = _post(x, mi, aggp, wstack, bstack, u, final)
        if final:
            outs.append(res[0])
        else:
            xo, x, mi, mj_table = res
            outs.append(xo)

    return jnp.stack(outs)[:, :n_atoms, :]


# speedup vs baseline: 1.0086x; 1.0086x over previous
import functools

import jax
import jax.numpy as jnp
import numpy as np
from jax import lax
from jax.experimental import pallas as pl
from jax.experimental.pallas import tpu as pltpu
from jax.experimental.pallas import tpu_sc as plsc

F = 128
N_RBF = 32
CUTOFF = 5.0
N_MODULES = 5
LOG2 = 0.6931471805599453

NC = 2
NS = 16
NW = NC * NS
CHUNK = 80


def _ssp(x):
    xx = jnp.abs(x) * -1.4426950408889634
    n = jnp.floor(xx + 0.5)
    fr = xx - n
    p = 1.535336188319500e-4
    p = p * fr + 1.339887440266574e-3
    p = p * fr + 9.618437357674640e-3
    p = p * fr + 5.550332471162809e-2
    p = p * fr + 2.402264791363012e-1
    p = p * fr + 6.931472028550421e-1
    p = p * fr + 1.0
    ei = jnp.clip(n, -127.0, 127.0).astype(jnp.int32)
    t = lax.bitcast_convert_type((ei + 127) << 23, jnp.float32) * p
    y = t / (2.0 + t)
    y2 = y * y
    l = 2.0 * y * (1.0 + y2 * (1.0 / 3.0 + y2 * (0.2 + y2 * (1.0 / 7.0))))
    return jnp.maximum(x, 0.0) + l - LOG2


def _rbf_gate_body(r_ref, wg_ref, g5_ref):
    r = r_ref[...]
    d2 = jnp.sum(r * r, axis=1, keepdims=True)
    d = jnp.sqrt(d2)
    step = CUTOFF / (N_RBF - 1)
    centers = lax.broadcasted_iota(jnp.int32, (1, N_RBF), 1).astype(jnp.float32) * step
    width = CUTOFF / N_RBF
    gamma = 0.5 / (width * width)
    nl2g = -gamma * 1.4426950408889634
    xx = nl2g * (d - centers) ** 2
    n = jnp.floor(xx + 0.5)
    fr = xx - n
    p = 1.535336188319500e-4
    p = p * fr + 1.339887440266574e-3
    p = p * fr + 9.618437357674640e-3
    p = p * fr + 5.550332471162809e-2
    p = p * fr + 2.402264791363012e-1
    p = p * fr + 6.931472028550421e-1
    p = p * fr + 1.0
    ei = jnp.clip(n, -127.0, 127.0).astype(jnp.int32)
    scale = lax.bitcast_convert_type((ei + 127) << 23, jnp.float32)
    phi = scale * p
    dc = jnp.clip(d, 0.0, CUTOFF)
    y = (dc * (jnp.pi / CUTOFF)) - (0.5 * jnp.pi)
    y2 = y * y
    sy = -1.9515295891e-4
    sy = sy * y2 + 8.3321608736e-3
    sy = sy * y2 - 1.6666654611e-1
    sy = sy * y2 + 1.0
    sy = sy * y
    fcut = 0.5 * (1.0 - sy)
    fcut = fcut * (d < CUTOFF).astype(jnp.float32)
    g = phi * fcut
    for m in range(N_MODULES):
        gm = jnp.dot(g, wg_ref[m], preferred_element_type=jnp.float32)
        u = lax.bitcast_convert_type(gm, jnp.uint32)
        half = u.shape[0] // 2
        w = (u[:half] >> 16) | (u[half:] & jnp.uint32(0xFFFF0000))
        g5_ref[m] = lax.bitcast_convert_type(w, jnp.int32)


def _rbf_gate(r_ij, wg5):
    e = r_ij.shape[0]
    blk = 1600
    grid = e // blk
    return pl.pallas_call(
        _rbf_gate_body,
        grid=(grid,),
        in_specs=[
            pl.BlockSpec((blk, 3), lambda i: (i, 0)),
            pl.BlockSpec((N_MODULES, N_RBF, F), lambda i: (0, 0, 0)),
        ],
        out_specs=pl.BlockSpec((N_MODULES, blk // 2, F), lambda i: (0, i, 0)),
        out_shape=jax.ShapeDtypeStruct((N_MODULES, e // 2, F), jnp.int32),
    )(r_ij, wg5)


def _embed_body(tab_hbm, idx_hbm, out_hbm, idx_v, rows_v, sem):
    c = lax.axis_index("c")
    s = lax.axis_index("s")
    wid = c * NS + s
    bpw = out_hbm.shape[0] // NW
    base = wid * bpw
    for k in range(bpw // CHUNK):
        off = base + k * CHUNK
        pltpu.sync_copy(idx_hbm.at[pl.ds(off, CHUNK)], idx_v)
        pltpu.async_copy(tab_hbm.at[idx_v], rows_v, sem).wait()
        pltpu.sync_copy(rows_v, out_hbm.at[pl.ds(off, CHUNK)])


def _embed_gather(table, idx_pad):
    a_pad = idx_pad.shape[0]
    mesh = plsc.VectorSubcoreMesh(core_axis_name="c", subcore_axis_name="s")
    fn = pl.kernel(
        _embed_body,
        out_type=jax.ShapeDtypeStruct((a_pad, F), jnp.float32),
        mesh=mesh,
        scratch_types=[
            pltpu.VMEM((CHUNK,), jnp.int32),
            pltpu.VMEM((CHUNK, F), jnp.float32),
            pltpu.SemaphoreType.DMA,
        ],
    )
    return fn(table, idx_pad)


RING = 2


def _edge_body(g_hbm, mj_hbm, ii_hbm, ij_hbm, z_hbm, out_hbm, *scr):
    ii = scr[0:RING]
    ij = scr[RING:2 * RING]
    gb = scr[2 * RING:3 * RING]
    pb = scr[3 * RING:4 * RING]
    acc = scr[4 * RING]
    sems = scr[4 * RING + 1:]
    sii = sems[0:RING]
    sij = sems[RING:2 * RING]
    sg = sems[2 * RING:3 * RING]
    sm = sems[3 * RING:4 * RING]
    ss = sems[4 * RING:5 * RING]

    c = lax.axis_index("c")
    s = lax.axis_index("s")
    wid = c * NS + s
    a_pad = acc.shape[0]
    rpt = a_pad // NS
    epw = ii_hbm.shape[0] // NW
    nch = epw // CHUNK
    base0 = wid * epw

    pltpu.sync_copy(z_hbm, acc.at[pl.ds(s * rpt, rpt)])
    plsc.subcore_barrier()

    def start(i, b):
        base = base0 + i * CHUNK
        pltpu.async_copy(ii_hbm.at[pl.ds(base, CHUNK)], ii[b], sii[b])
        pltpu.async_copy(ij_hbm.at[pl.ds(base, CHUNK)], ij[b], sij[b])
        gbase = pl.multiple_of(base // 2, 8)
        pltpu.async_copy(g_hbm.at[pl.ds(gbase, CHUNK // 2)], gb[b], sg[b])

    def fire(b):
        pltpu.make_async_copy(ij_hbm.at[pl.ds(0, CHUNK)], ij[b], sij[b]).wait()
        pltpu.async_copy(mj_hbm.at[ij[b]], pb[b], sm[b])

    def wait_scatter(b):
        pltpu.make_async_copy(pb[b], acc.at[ii[b]], ss[b]).wait()

    def proc(b):
        pltpu.make_async_copy(g_hbm.at[pl.ds(0, CHUNK // 2)], gb[b], sg[b]).wait()
        pltpu.make_async_copy(mj_hbm.at[ij[b]], pb[b], sm[b]).wait()
        pltpu.make_async_copy(ii_hbm.at[pl.ds(0, CHUNK)], ii[b], sii[b]).wait()

        @plsc.parallel_loop(0, CHUNK // 2, 1, unroll=4)
        def row_fn(r2):
            e = r2 * 2
            for jj in range(F // 16):
                sl = pl.ds(16 * jj, 16)
                wv = gb[b][r2, sl]
                ga = lax.bitcast_convert_type(
                    lax.shift_left(wv, 16), jnp.float32)
                gc = lax.bitcast_convert_type(
                    lax.bitwise_and(wv, jnp.int32(-65536)), jnp.float32)
                pb[b][e, sl] = ga * pb[b][e, sl]
                pb[b][e + 1, sl] = gc * pb[b][e + 1, sl]
        pltpu.async_copy(pb[b], acc.at[ii[b]], ss[b], add=True)

    start(0, 0)
    fire(0)
    start(1, 1)
    fire(1)
    proc(0)

    def step(i, carry):
        b = lax.rem(i, 2)

        def do(b, o):
            wait_scatter(o)
            start(i + 1, o)
            fire(o)
            proc(b)
        lax.cond(b == 0, lambda: do(0, 1), lambda: do(1, 0))
        return carry
    lax.fori_loop(1, nch - 1, step, 0)

    bl = (nch - 1) % 2
    wait_scatter(1 - bl)
    proc(bl)
    wait_scatter(bl)

    plsc.subcore_barrier()
    pltpu.sync_copy(acc.at[pl.ds(s * rpt, rpt)],
                    out_hbm.at[c, pl.ds(s * rpt, rpt)])


def _edge_stage(g, mj_table, idx_i, idx_j, zrows):
    a_pad = mj_table.shape[0]
    mesh = plsc.VectorSubcoreMesh(core_axis_name="c", subcore_axis_name="s")
    fn = pl.kernel(
        _edge_body,
        out_type=jax.ShapeDtypeStruct((NC, a_pad, F), jnp.float32),
        mesh=mesh,
        scratch_types=(
            [pltpu.VMEM((CHUNK,), jnp.int32) for _ in range(2 * RING)]
            + [pltpu.VMEM((CHUNK // 2, F), jnp.int32) for _ in range(RING)]
            + [pltpu.VMEM((CHUNK, F), jnp.float32) for _ in range(RING)]
            + [pltpu.VMEM_SHARED((a_pad, F), jnp.float32)]
            + [pltpu.SemaphoreType.DMA for _ in range(5 * RING)]
        ),
    )
    return fn(g, mj_table, idx_i, idx_j, zrows)


def _pre_body(x_ref, w_ref, b_ref, mi_ref, mj_ref):
    xt = _ssp(x_ref[...])
    mi_ref[...] = jnp.dot(xt, w_ref[0], preferred_element_type=jnp.float32) + b_ref[0]
    mj_ref[...] = jnp.dot(xt, w_ref[1], preferred_element_type=jnp.float32) + b_ref[1]


def _pre(x, w2, b2):
    a_pad = x.shape[0]
    blk = 512
    grid = a_pad // blk
    sds = jax.ShapeDtypeStruct((a_pad, F), jnp.float32)
    return pl.pallas_call(
        _pre_body,
        grid=(grid,),
        in_specs=[
            pl.BlockSpec((blk, F), lambda i: (i, 0)),
            pl.BlockSpec((2, F, F), lambda i: (0, 0, 0)),
            pl.BlockSpec((2, 1, F), lambda i: (0, 0, 0)),
        ],
        out_specs=[pl.BlockSpec((blk, F), lambda i: (i, 0))] * 2,
        out_shape=[sds, sds],
    )(x, w2, b2)


def _post_body(final, x_ref, mi_ref, agg_ref, w_ref, b_ref, u_ref, *outs):
    def d(h, k):
        return jnp.dot(_ssp(h), w_ref[k], preferred_element_type=jnp.float32) + b_ref[k]

    v = mi_ref[...] + agg_ref[0] + agg_ref[1]
    k = 0
    for _ in range(3):
        v = v + d(d(v, k), k + 1)
        k += 2
    v = d(v, k)
    k += 1
    x = u_ref[...] * x_ref[...] + v
    for _ in range(2):
        x = x + d(d(x, k), k + 1)
        k += 2
    xo = x
    for _ in range(1):
        xo = xo + d(d(xo, k), k + 1)
        k += 2
    outs[0][...] = d(xo, k)
    k += 1
    if not final:
        outs[1][...] = x
        xt = _ssp(x)
        outs[2][...] = jnp.dot(xt, w_ref[k], preferred_element_type=jnp.float32) + b_ref[k]
        outs[3][...] = jnp.dot(xt, w_ref[k + 1], preferred_element_type=jnp.float32) + b_ref[k + 1]


def _post(x, mi, aggp, wstack, bstack, u, final):
    a_pad = x.shape[0]
    blk = 512
    grid = a_pad // blk
    nw = wstack.shape[0]
    sds = jax.ShapeDtypeStruct((a_pad, F), jnp.float32)
    out_shape = [sds] if final else [sds, sds, sds, sds]
    xspec = pl.BlockSpec((blk, F), lambda i: (i, 0))
    return pl.pallas_call(
        functools.partial(_post_body, final),
        grid=(grid,),
        in_specs=[
            xspec, xspec,
            pl.BlockSpec((2, blk, F), lambda i: (0, i, 0)),
            pl.BlockSpec((nw, F, F), lambda i: (0, 0, 0)),
            pl.BlockSpec((nw, 1, F), lambda i: (0, 0, 0)),
            pl.BlockSpec((1, F), lambda i: (0, 0)),
        ],
        out_specs=[xspec] * len(out_shape),
        out_shape=out_shape,
    )(x, mi, aggp, wstack, bstack, u)


def _stack_post_weights(params, m):
    p = params[m]
    ws, bs = [], []
    for rp in p['int_res']:
        ws += [rp['W1'], rp['W2']]
        bs += [rp['b1'], rp['b2']]
    ws.append(p['Wv'])
    bs.append(p['bv'])
    for rp in p['atom_res']:
        ws += [rp['W1'], rp['W2']]
        bs += [rp['b1'], rp['b2']]
    for rp in p['out_res']:
        ws += [rp['W1'], rp['W2']]
        bs += [rp['b1'], rp['b2']]
    ws.append(p['Wout'])
    bs.append(p['bout'])
    if m + 1 < len(params):
        ws += [params[m + 1]['Wi'], params[m + 1]['Wj']]
        bs += [params[m + 1]['bi'], params[m + 1]['bj']]
    wstack = jnp.stack(ws)
    bstack = jnp.stack(bs)[:, None, :]
    return wstack, bstack, p['u'][None, :]


def kernel(atomic_numbers, r_ij, idx_i, idx_j, idx_m, embedding, params):
    n_atoms = atomic_numbers.shape[0]
    a_pad = ((n_atoms + NW * CHUNK - 1) // (NW * CHUNK)) * (NW * CHUNK)

    an_pad = jnp.concatenate([
        atomic_numbers.astype(jnp.int32),
        jnp.zeros((a_pad - n_atoms,), jnp.int32),
    ])
    n_edges = idx_i.shape[0]
    eblk = 1600
    _rr = np.arange(eblk // 2)
    _bases = (np.arange(n_edges // eblk) * eblk)[:, None]
    _inter = np.empty((n_edges // eblk, eblk), np.int32)
    _inter[:, 0::2] = _bases + _rr
    _inter[:, 1::2] = _bases + eblk // 2 + _rr
    eperm = _inter.reshape(-1)
    idx_i = idx_i.astype(jnp.int32)[eperm]
    idx_j = idx_j.astype(jnp.int32)[eperm]
    zrows = jnp.zeros((a_pad // NS, F), jnp.float32)

    wg5 = jnp.stack([p['Wg'] for p in params])
    g5 = _rbf_gate(r_ij, wg5)

    x = _embed_gather(embedding, an_pad)
    w2 = jnp.stack([params[0]['Wi'], params[0]['Wj']])
    b2 = jnp.stack([params[0]['bi'], params[0]['bj']])[:, None, :]
    mi, mj_table = _pre(x, w2, b2)

    outs = []
    for m in range(N_MODULES):
        aggp = _edge_stage(g5[m], mj_table, idx_i, idx_j, zrows)
        final = m == N_MODULES - 1
        wstack, bstack, u = _stack_post_weights(params, m)
        res = _post(x, mi, aggp, wstack, bstack, u, final)
        if final:
            outs.append(res[0])
        else:
            xo, x, mi, mj_table = res
            outs.append(xo)

    return jnp.stack(outs)[:, :n_atoms, :]


# speedup vs baseline: 1.1213x; 1.1117x over previous
import functools

import jax
import jax.numpy as jnp
import numpy as np
from jax import lax
from jax.experimental import pallas as pl
from jax.experimental.pallas import tpu as pltpu
from jax.experimental.pallas import tpu_sc as plsc

F = 128
N_RBF = 32
CUTOFF = 5.0
N_MODULES = 5
LOG2 = 0.6931471805599453

NC = 2
NS = 16
NW = NC * NS
CHUNK = 80


def _ssp(x):
    return jax.nn.softplus(x) - LOG2


def _rbf_gate_body(r_ref, wg_ref, g5_ref):
    r = r_ref[...]
    d2 = jnp.sum(r * r, axis=1, keepdims=True)
    d = jnp.sqrt(d2)
    step = CUTOFF / (N_RBF - 1)
    centers = lax.broadcasted_iota(jnp.int32, (1, N_RBF), 1).astype(jnp.float32) * step
    width = CUTOFF / N_RBF
    gamma = 0.5 / (width * width)
    nl2g = -gamma * 1.4426950408889634
    xx = nl2g * (d - centers) ** 2
    n = jnp.floor(xx + 0.5)
    fr = xx - n
    p = 1.535336188319500e-4
    p = p * fr + 1.339887440266574e-3
    p = p * fr + 9.618437357674640e-3
    p = p * fr + 5.550332471162809e-2
    p = p * fr + 2.402264791363012e-1
    p = p * fr + 6.931472028550421e-1
    p = p * fr + 1.0
    ei = jnp.clip(n, -127.0, 127.0).astype(jnp.int32)
    scale = lax.bitcast_convert_type((ei + 127) << 23, jnp.float32)
    phi = scale * p
    dc = jnp.clip(d, 0.0, CUTOFF)
    y = (dc * (jnp.pi / CUTOFF)) - (0.5 * jnp.pi)
    y2 = y * y
    sy = -1.9515295891e-4
    sy = sy * y2 + 8.3321608736e-3
    sy = sy * y2 - 1.6666654611e-1
    sy = sy * y2 + 1.0
    sy = sy * y
    fcut = 0.5 * (1.0 - sy)
    fcut = fcut * (d < CUTOFF).astype(jnp.float32)
    g = phi * fcut
    for m in range(N_MODULES):
        gm = jnp.dot(g, wg_ref[m], preferred_element_type=jnp.float32)
        u = lax.bitcast_convert_type(gm, jnp.uint32)
        half = u.shape[0] // 2
        w = (u[:half] >> 16) | (u[half:] & jnp.uint32(0xFFFF0000))
        g5_ref[m] = lax.bitcast_convert_type(w, jnp.int32)


def _rbf_gate(r_ij, wg5):
    e = r_ij.shape[0]
    blk = 1600
    grid = e // blk
    return pl.pallas_call(
        _rbf_gate_body,
        grid=(grid,),
        in_specs=[
            pl.BlockSpec((blk, 3), lambda i: (i, 0)),
            pl.BlockSpec((N_MODULES, N_RBF, F), lambda i: (0, 0, 0)),
        ],
        out_specs=pl.BlockSpec((N_MODULES, blk // 2, F), lambda i: (0, i, 0)),
        out_shape=jax.ShapeDtypeStruct((N_MODULES, e // 2, F), jnp.int32),
    )(r_ij, wg5)


def _embed_body(tab_hbm, idx_hbm, out_hbm, idx_v, rows_v, sem):
    c = lax.axis_index("c")
    s = lax.axis_index("s")
    wid = c * NS + s
    bpw = out_hbm.shape[0] // NW
    base = wid * bpw
    for k in range(bpw // CHUNK):
        off = base + k * CHUNK
        pltpu.sync_copy(idx_hbm.at[pl.ds(off, CHUNK)], idx_v)
        pltpu.async_copy(tab_hbm.at[idx_v], rows_v, sem).wait()
        pltpu.sync_copy(rows_v, out_hbm.at[pl.ds(off, CHUNK)])


def _embed_gather(table, idx_pad):
    a_pad = idx_pad.shape[0]
    mesh = plsc.VectorSubcoreMesh(core_axis_name="c", subcore_axis_name="s")
    fn = pl.kernel(
        _embed_body,
        out_type=jax.ShapeDtypeStruct((a_pad, F), jnp.float32),
        mesh=mesh,
        scratch_types=[
            pltpu.VMEM((CHUNK,), jnp.int32),
            pltpu.VMEM((CHUNK, F), jnp.float32),
            pltpu.SemaphoreType.DMA,
        ],
    )
    return fn(table, idx_pad)


RING = 2


def _edge_body(g_hbm, mj_hbm, ii_hbm, ij_hbm, z_hbm, out_hbm, *scr):
    ii = scr[0:RING]
    ij = scr[RING:2 * RING]
    gb = scr[2 * RING:3 * RING]
    pb = scr[3 * RING:4 * RING]
    acc = scr[4 * RING]
    sems = scr[4 * RING + 1:]
    sii = sems[0:RING]
    sij = sems[RING:2 * RING]
    sg = sems[2 * RING:3 * RING]
    sm = sems[3 * RING:4 * RING]
    ss = sems[4 * RING:5 * RING]

    c = lax.axis_index("c")
    s = lax.axis_index("s")
    wid = c * NS + s
    a_pad = acc.shape[0]
    rpt = a_pad // NS
    epw = ii_hbm.shape[0] // NW
    nch = epw // CHUNK
    base0 = wid * epw

    pltpu.sync_copy(z_hbm, acc.at[pl.ds(s * rpt, rpt)])
    plsc.subcore_barrier()

    def start(i, b):
        base = base0 + i * CHUNK
        pltpu.async_copy(ii_hbm.at[pl.ds(base, CHUNK)], ii[b], sii[b])
        pltpu.async_copy(ij_hbm.at[pl.ds(base, CHUNK)], ij[b], sij[b])
        gbase = pl.multiple_of(base // 2, 8)
        pltpu.async_copy(g_hbm.at[pl.ds(gbase, CHUNK // 2)], gb[b], sg[b])

    def fire(b):
        pltpu.make_async_copy(ij_hbm.at[pl.ds(0, CHUNK)], ij[b], sij[b]).wait()
        pltpu.async_copy(mj_hbm.at[ij[b]], pb[b], sm[b])

    def wait_scatter(b):
        pltpu.make_async_copy(pb[b], acc.at[ii[b]], ss[b]).wait()

    def proc(b):
        pltpu.make_async_copy(g_hbm.at[pl.ds(0, CHUNK // 2)], gb[b], sg[b]).wait()
        pltpu.make_async_copy(mj_hbm.at[ij[b]], pb[b], sm[b]).wait()
        pltpu.make_async_copy(ii_hbm.at[pl.ds(0, CHUNK)], ii[b], sii[b]).wait()

        @plsc.parallel_loop(0, CHUNK // 2, 1, unroll=4)
        def row_fn(r2):
            e = r2 * 2
            for jj in range(F // 16):
                sl = pl.ds(16 * jj, 16)
                wv = gb[b][r2, sl]
                ga = lax.bitcast_convert_type(
                    lax.shift_left(wv, 16), jnp.float32)
                gc = lax.bitcast_convert_type(
                    lax.bitwise_and(wv, jnp.int32(-65536)), jnp.float32)
                pb[b][e, sl] = ga * pb[b][e, sl]
                pb[b][e + 1, sl] = gc * pb[b][e + 1, sl]
        pltpu.async_copy(pb[b], acc.at[ii[b]], ss[b], add=True)

    start(0, 0)
    fire(0)
    start(1, 1)
    fire(1)
    proc(0)

    def step(i, carry):
        b = lax.rem(i, 2)

        def do(b, o):
            wait_scatter(o)
            start(i + 1, o)
            fire(o)
            proc(b)
        lax.cond(b == 0, lambda: do(0, 1), lambda: do(1, 0))
        return carry
    lax.fori_loop(1, nch - 1, step, 0)

    bl = (nch - 1) % 2
    wait_scatter(1 - bl)
    proc(bl)
    wait_scatter(bl)

    plsc.subcore_barrier()
    pltpu.sync_copy(acc.at[pl.ds(s * rpt, rpt)],
                    out_hbm.at[c, pl.ds(s * rpt, rpt)])


def _edge_stage(g, mj_table, idx_i, idx_j, zrows):
    a_pad = mj_table.shape[0]
    mesh = plsc.VectorSubcoreMesh(core_axis_name="c", subcore_axis_name="s")
    fn = pl.kernel(
        _edge_body,
        out_type=jax.ShapeDtypeStruct((NC, a_pad, F), jnp.float32),
        mesh=mesh,
        scratch_types=(
            [pltpu.VMEM((CHUNK,), jnp.int32) for _ in range(2 * RING)]
            + [pltpu.VMEM((CHUNK // 2, F), jnp.int32) for _ in range(RING)]
            + [pltpu.VMEM((CHUNK, F), jnp.float32) for _ in range(RING)]
            + [pltpu.VMEM_SHARED((a_pad, F), jnp.float32)]
            + [pltpu.SemaphoreType.DMA for _ in range(5 * RING)]
        ),
    )
    return fn(g, mj_table, idx_i, idx_j, zrows)


def _pre_body(x_ref, w_ref, b_ref, mi_ref, mj_ref):
    xt = _ssp(x_ref[...])
    mi_ref[...] = jnp.dot(xt, w_ref[0], preferred_element_type=jnp.float32) + b_ref[0]
    mj_ref[...] = jnp.dot(xt, w_ref[1], preferred_element_type=jnp.float32) + b_ref[1]


def _pre(x, w2, b2):
    a_pad = x.shape[0]
    blk = 512
    grid = a_pad // blk
    sds = jax.ShapeDtypeStruct((a_pad, F), jnp.float32)
    return pl.pallas_call(
        _pre_body,
        grid=(grid,),
        in_specs=[
            pl.BlockSpec((blk, F), lambda i: (i, 0)),
            pl.BlockSpec((2, F, F), lambda i: (0, 0, 0)),
            pl.BlockSpec((2, 1, F), lambda i: (0, 0, 0)),
        ],
        out_specs=[pl.BlockSpec((blk, F), lambda i: (i, 0))] * 2,
        out_shape=[sds, sds],
    )(x, w2, b2)


def _post_body(final, x_ref, mi_ref, agg_ref, w_ref, b_ref, u_ref, *outs):
    def d(h, k):
        return jnp.dot(_ssp(h), w_ref[k], preferred_element_type=jnp.float32) + b_ref[k]

    v = mi_ref[...] + agg_ref[0] + agg_ref[1]
    k = 0
    for _ in range(3):
        v = v + d(d(v, k), k + 1)
        k += 2
    v = d(v, k)
    k += 1
    x = u_ref[...] * x_ref[...] + v
    for _ in range(2):
        x = x + d(d(x, k), k + 1)
        k += 2
    xo = x
    for _ in range(1):
        xo = xo + d(d(xo, k), k + 1)
        k += 2
    outs[0][...] = d(xo, k)
    k += 1
    if not final:
        outs[1][...] = x
        xt = _ssp(x)
        outs[2][...] = jnp.dot(xt, w_ref[k], preferred_element_type=jnp.float32) + b_ref[k]
        outs[3][...] = jnp.dot(xt, w_ref[k + 1], preferred_element_type=jnp.float32) + b_ref[k + 1]


def _post(x, mi, aggp, wstack, bstack, u, final):
    a_pad = x.shape[0]
    blk = 512
    grid = a_pad // blk
    nw = wstack.shape[0]
    sds = jax.ShapeDtypeStruct((a_pad, F), jnp.float32)
    out_shape = [sds] if final else [sds, sds, sds, sds]
    xspec = pl.BlockSpec((blk, F), lambda i: (i, 0))
    return pl.pallas_call(
        functools.partial(_post_body, final),
        grid=(grid,),
        in_specs=[
            xspec, xspec,
            pl.BlockSpec((2, blk, F), lambda i: (0, i, 0)),
            pl.BlockSpec((nw, F, F), lambda i: (0, 0, 0)),
            pl.BlockSpec((nw, 1, F), lambda i: (0, 0, 0)),
            pl.BlockSpec((1, F), lambda i: (0, 0)),
        ],
        out_specs=[xspec] * len(out_shape),
        out_shape=out_shape,
    )(x, mi, aggp, wstack, bstack, u)


def _stack_post_weights(params, m):
    p = params[m]
    ws, bs = [], []
    for rp in p['int_res']:
        ws += [rp['W1'], rp['W2']]
        bs += [rp['b1'], rp['b2']]
    ws.append(p['Wv'])
    bs.append(p['bv'])
    for rp in p['atom_res']:
        ws += [rp['W1'], rp['W2']]
        bs += [rp['b1'], rp['b2']]
    for rp in p['out_res']:
        ws += [rp['W1'], rp['W2']]
        bs += [rp['b1'], rp['b2']]
    ws.append(p['Wout'])
    bs.append(p['bout'])
    if m + 1 < len(params):
        ws += [params[m + 1]['Wi'], params[m + 1]['Wj']]
        bs += [params[m + 1]['bi'], params[m + 1]['bj']]
    wstack = jnp.stack(ws)
    bstack = jnp.stack(bs)[:, None, :]
    return wstack, bstack, p['u'][None, :]


def kernel(atomic_numbers, r_ij, idx_i, idx_j, idx_m, embedding, params):
    n_atoms = atomic_numbers.shape[0]
    a_pad = ((n_atoms + NW * CHUNK - 1) // (NW * CHUNK)) * (NW * CHUNK)

    an_pad = jnp.concatenate([
        atomic_numbers.astype(jnp.int32),
        jnp.zeros((a_pad - n_atoms,), jnp.int32),
    ])
    n_edges = idx_i.shape[0]
    eblk = 1600
    _rr = np.arange(eblk // 2)
    _bases = (np.arange(n_edges // eblk) * eblk)[:, None]
    _inter = np.empty((n_edges // eblk, eblk), np.int32)
    _inter[:, 0::2] = _bases + _rr
    _inter[:, 1::2] = _bases + eblk // 2 + _rr
    eperm = _inter.reshape(-1)
    idx_i = idx_i.astype(jnp.int32)[eperm]
    idx_j = idx_j.astype(jnp.int32)[eperm]
    zrows = jnp.zeros((a_pad // NS, F), jnp.float32)

    wg5 = jnp.stack([p['Wg'] for p in params])
    g5 = _rbf_gate(r_ij, wg5)

    x = _embed_gather(embedding, an_pad)
    w2 = jnp.stack([params[0]['Wi'], params[0]['Wj']])
    b2 = jnp.stack([params[0]['bi'], params[0]['bj']])[:, None, :]
    mi, mj_table = _pre(x, w2, b2)

    outs = []
    for m in range(N_MODULES):
        aggp = _edge_stage(g5[m], mj_table, idx_i, idx_j, zrows)
        final = m == N_MODULES - 1
        wstack, bstack, u = _stack_post_weights(params, m)
        res = _post(x, mi, aggp, wstack, bstack, u, final)
        if final:
            outs.append(res[0])
        else:
            xo, x, mi, mj_table = res
            outs.append(xo)

    return jnp.stack(outs)[:, :n_atoms, :]
